# baseline (speedup 1.0000x reference)
import functools

import jax
import jax.numpy as jnp
from jax import lax
from jax.experimental import pallas as pl
from jax.experimental.pallas import tpu as pltpu
from jax.experimental.pallas import tpu_sc as plsc

N = 10000
NP = 10240
E = 320000
EP = 322560
H = 128
G = 128
C = 72
EPT = EP // 32
EGROUPS = EPT // C
NGTOT = EP // C
RT = NP // 16
NT = NP // 32
NC = 2
GA = 2 * G
PB = 5

_mesh = plsc.VectorSubcoreMesh(core_axis_name="c", subcore_axis_name="s")


def _wid():
    return lax.axis_index("s") * NC + lax.axis_index("c")


@functools.partial(
    pl.kernel,
    mesh=_mesh,
    out_type=[
        jax.ShapeDtypeStruct((2 * NP,), jnp.float32),
        jax.ShapeDtypeStruct((2 * G,), jnp.float32),
    ],
    scratch_types=[
        pltpu.VMEM((2, 2, C), jnp.int32),
        pltpu.VMEM((2, C), jnp.float32),
        pltpu.VMEM((64,), jnp.int32),
        pltpu.VMEM((64,), jnp.float32),
        pltpu.VMEM_SHARED((NP,), jnp.float32),
        pltpu.VMEM_SHARED((GA,), jnp.float32),
        pltpu.SemaphoreType.DMA((2,)),
        pltpu.SemaphoreType.DMA((2,)),
        pltpu.SemaphoreType.DMA((2,)),
    ],
)
def _deg_cnt(edata_h, attr2_h, batch_h, z1_h, deg_out, cnt_out,
             ebuf, abuf, nidx_v, nval_v, deg_acc, cnt_acc, esem, asem, ssem):
    cid = lax.axis_index("c")
    sid = lax.axis_index("s")
    wid = _wid()
    pltpu.sync_copy(z1_h, deg_acc.at[pl.ds(sid * RT, RT)])

    @pl.when(sid == 0)
    def _():
        pltpu.sync_copy(z1_h.at[pl.ds(0, GA)], cnt_acc)

    plsc.subcore_barrier()

    def dissue(g, b):
        pltpu.async_copy(edata_h.at[wid * EGROUPS + g], ebuf.at[b], esem.at[b])
        pltpu.async_copy(attr2_h.at[wid * EGROUPS + g], abuf.at[b], asem.at[b])

    dissue(0, 0)

    def ebody(g, carry):
        b = lax.rem(g, 2)
        nb = 1 - b

        @pl.when(g + 1 < EGROUPS)
        def _():
            @pl.when(g >= 1)
            def _():
                pltpu.make_async_copy(
                    abuf.at[nb], deg_acc.at[ebuf.at[nb, 1]], ssem.at[nb]
                ).wait()
            dissue(g + 1, nb)

        pltpu.make_async_copy(edata_h.at[wid * EGROUPS + g],
                              ebuf.at[b], esem.at[b]).wait()
        pltpu.make_async_copy(attr2_h.at[wid * EGROUPS + g],
                              abuf.at[b], asem.at[b]).wait()
        pltpu.async_copy(abuf.at[b], deg_acc.at[ebuf.at[b, 1]],
                         ssem.at[b], add=True)
        return carry

    lax.fori_loop(0, EGROUPS, ebody, 0)
    for gl in (EGROUPS - 2, EGROUPS - 1):
        b = gl % 2
        pltpu.make_async_copy(abuf.at[b], deg_acc.at[ebuf.at[b, 1]],
                              ssem.at[b]).wait()

    for k in range(64 // 16):
        nval_v[pl.ds(k * 16, 16)] = jnp.full((16,), 1.0, jnp.float32)
    for g in range(NT // 64):
        nbase = wid * NT + g * 64
        pltpu.sync_copy(batch_h.at[pl.ds(nbase, 64)], nidx_v)
        pltpu.sync_copy(nval_v, cnt_acc.at[nidx_v], add=True)

    plsc.subcore_barrier()
    pltpu.sync_copy(deg_acc.at[pl.ds(sid * RT, RT)],
                    deg_out.at[pl.ds(cid * NP + sid * RT, RT)])

    @pl.when(sid == 0)
    def _():
        pltpu.sync_copy(cnt_acc.at[pl.ds(0, G)], cnt_out.at[pl.ds(cid * G, G)])


@functools.partial(
    pl.kernel,
    mesh=_mesh,
    out_type=jax.ShapeDtypeStruct((2 * NP, H), jnp.float32),
    scratch_types=[
        pltpu.VMEM((PB, 2, C), jnp.int32),
        pltpu.VMEM((PB, C * 16), jnp.float32),
        pltpu.VMEM((2, C, H), jnp.float32),
        pltpu.VMEM((2, C, H), jnp.float32),
        pltpu.VMEM_SHARED((NP, H), jnp.float32),
        pltpu.SemaphoreType.DMA((PB,)),
        pltpu.SemaphoreType.DMA((PB,)),
        pltpu.SemaphoreType.DMA((2,)),
        pltpu.SemaphoreType.DMA((2,)),
    ],
)
def _agg(hs_h, edata_h, arep_h, z2_h, out_h,
         ebuf, awbuf, rows_g, rows_s, acc, esem, wsem, gsem, ssem):
    cid = lax.axis_index("c")
    sid = lax.axis_index("s")
    wid = _wid()
    pltpu.sync_copy(z2_h, acc.at[pl.ds(sid * RT, RT)])
    plsc.subcore_barrier()

    def fetch(g, m):
        pltpu.async_copy(edata_h.at[wid * EGROUPS + g], ebuf.at[m], esem.at[m])
        pltpu.async_copy(arep_h.at[wid * EGROUPS + g], awbuf.at[m], wsem.at[m])

    def wait_fetch_e(g, m):
        pltpu.make_async_copy(edata_h.at[wid * EGROUPS + g], ebuf.at[m],
                              esem.at[m]).wait()

    def gather(m, r):
        pltpu.async_copy(hs_h.at[ebuf.at[m, 0]], rows_g.at[r], gsem.at[r])

    def wait_scatter(m, r):
        pltpu.make_async_copy(rows_s.at[r], acc.at[ebuf.at[m, 1]],
                              ssem.at[r]).wait()

    for g in range(3):
        fetch(g, g)
    for g in range(2):
        wait_fetch_e(g, g)
        gather(g, g)

    def body(g, carry):
        b2 = lax.rem(g, 2)
        b5 = lax.rem(g, PB)
        p5 = lax.rem(g + 2, PB)
        f5 = lax.rem(g + 3, PB)

        pltpu.make_async_copy(arep_h.at[wid * EGROUPS + g], awbuf.at[b5],
                              wsem.at[b5]).wait()
        pltpu.make_async_copy(hs_h.at[ebuf.at[b5, 0]], rows_g.at[b2],
                              gsem.at[b2]).wait()

        def _mul(e, mc):
            w16 = awbuf[b5, pl.ds(e * 16, 16)]
            for j in range(H // 16):
                sl = pl.ds(j * 16, 16)
                rows_s[b2, e, sl] = rows_g[b2, e, sl] * w16
            return mc

        lax.fori_loop(0, C, _mul, 0, unroll=8)

        @pl.when(g >= 2)
        def _():
            pb5 = lax.rem(g - 2, PB)
            wait_scatter(pb5, b2)

        pltpu.async_copy(rows_s.at[b2], acc.at[ebuf.at[b5, 1]],
                         ssem.at[b2], add=True)

        @pl.when(g + 2 < EGROUPS)
        def _():
            wait_fetch_e(g + 2, p5)
            gather(p5, b2)

        @pl.when(g + 3 < EGROUPS)
        def _():
            fetch(g + 3, f5)

        return carry

    lax.fori_loop(0, EGROUPS, body, 0)
    for gl in (EGROUPS - 2, EGROUPS - 1):
        wait_scatter(gl % PB, gl % 2)
    plsc.subcore_barrier()
    pltpu.sync_copy(acc.at[pl.ds(sid * RT, RT)],
                    out_h.at[pl.ds(cid * NP + sid * RT, RT)])


@functools.partial(
    pl.kernel,
    mesh=_mesh,
    out_type=jax.ShapeDtypeStruct((2 * G, H), jnp.float32),
    scratch_types=[
        pltpu.VMEM((64,), jnp.int32),
        pltpu.VMEM((64, H), jnp.float32),
        pltpu.VMEM_SHARED((GA, H), jnp.float32),
    ],
)
def _pool(x_h, batch_h, z2_h, out_h, nidx_v, rows_v, acc):
    cid = lax.axis_index("c")
    sid = lax.axis_index("s")
    wid = _wid()

    @pl.when(sid == 0)
    def _():
        pltpu.sync_copy(z2_h.at[pl.ds(0, GA)], acc)

    plsc.subcore_barrier()
    for g in range(NT // 64):
        nbase = wid * NT + g * 64
        pltpu.sync_copy(batch_h.at[pl.ds(nbase, 64)], nidx_v)
        pltpu.sync_copy(x_h.at[pl.ds(nbase, 64)], rows_v)
        pltpu.sync_copy(rows_v, acc.at[nidx_v], add=True)
    plsc.subcore_barrier()

    @pl.when(sid == 0)
    def _():
        pltpu.sync_copy(acc.at[pl.ds(0, G)], out_h.at[pl.ds(cid * G, G)])


_B = 512
_GRID = NP // _B


def _tc1_body(x_ref, we_ref, be_ref, w1_ref, rsq_ref, o_ref):
    h0 = jnp.dot(x_ref[...], we_ref[...],
                 preferred_element_type=jnp.float32) + be_ref[...]
    hw = jnp.dot(h0, w1_ref[...], preferred_element_type=jnp.float32)
    o_ref[...] = hw * rsq_ref[...]


def _tc2_body(p_ref, rsq_ref, b1_ref, w2_ref, o_ref):
    p = p_ref[...]
    s = (p[0] + p[1]) * rsq_ref[...] + b1_ref[...]
    h1 = jnp.maximum(s, 0.0)
    o_ref[...] = jnp.dot(h1, w2_ref[...],
                         preferred_element_type=jnp.float32) * rsq_ref[...]


def _tc3_body(p_ref, rsq_ref, b2_ref, o_ref):
    p = p_ref[...]
    s = (p[0] + p[1]) * rsq_ref[...] + b2_ref[...]
    o = jnp.maximum(s, 0.0)
    row = (pl.program_id(0) * _B
           + lax.broadcasted_iota(jnp.int32, (_B, 1), 0))
    o_ref[...] = jnp.where(row < N, o, 0.0)


def _tc1(x_p, W_enc, b_enc, W1, rsq):
    return pl.pallas_call(
        _tc1_body,
        grid=(_GRID,),
        in_specs=[
            pl.BlockSpec((_B, H), lambda g: (g, 0)),
            pl.BlockSpec((H, H), lambda g: (0, 0)),
            pl.BlockSpec((1, H), lambda g: (0, 0)),
            pl.BlockSpec((H, H), lambda g: (0, 0)),
            pl.BlockSpec((_B, 1), lambda g: (g, 0)),
        ],
        out_specs=pl.BlockSpec((_B, H), lambda g: (g, 0)),
        out_shape=jax.ShapeDtypeStruct((NP, H), jnp.float32),
    )(x_p, W_enc, b_enc, W1, rsq)


def _tc2(p, rsq, b1, W2):
    return pl.pallas_call(
        _tc2_body,
        grid=(_GRID,),
        in_specs=[
            pl.BlockSpec((2, _B, H), lambda g: (0, g, 0)),
            pl.BlockSpec((_B, 1), lambda g: (g, 0)),
            pl.BlockSpec((1, H), lambda g: (0, 0)),
            pl.BlockSpec((H, H), lambda g: (0, 0)),
        ],
        out_specs=pl.BlockSpec((_B, H), lambda g: (g, 0)),
        out_shape=jax.ShapeDtypeStruct((NP, H), jnp.float32),
    )(p, rsq, b1, W2)


def _tc3(p, rsq, b2):
    return pl.pallas_call(
        _tc3_body,
        grid=(_GRID,),
        in_specs=[
            pl.BlockSpec((2, _B, H), lambda g: (0, g, 0)),
            pl.BlockSpec((_B, 1), lambda g: (g, 0)),
            pl.BlockSpec((1, H), lambda g: (0, 0)),
        ],
        out_specs=pl.BlockSpec((_B, H), lambda g: (g, 0)),
        out_shape=jax.ShapeDtypeStruct((NP, H), jnp.float32),
    )(p, rsq, b2)


def kernel(x, edge_index, edge_attr, batch, W_enc, b_enc, W1, b1, W2, b2):
    f32 = jnp.float32
    i32 = jnp.int32
    src_p = jnp.concatenate([edge_index[0].astype(i32),
                             jnp.zeros((EP - E,), i32)])
    dst_p = jnp.concatenate([edge_index[1].astype(i32),
                             jnp.zeros((EP - E,), i32)])
    attr_p = jnp.concatenate([edge_attr.astype(f32), jnp.zeros((EP - E,), f32)])
    edata = jnp.stack(
        [src_p.reshape(NGTOT, C), dst_p.reshape(NGTOT, C)], axis=1)
    attr2 = attr_p.reshape(NGTOT, C)
    arep = jnp.broadcast_to(attr_p[:, None], (EP, 16)).reshape(NGTOT, C * 16)
    batch_p = jnp.concatenate([batch.astype(i32),
                               jnp.full((NP - N,), G, i32)])
    x_p = jnp.pad(x.astype(f32), ((0, NP - N), (0, 0)))
    z1 = jnp.zeros((RT,), f32)
    z2 = jnp.zeros((RT, H), f32)

    deg_flat, cnt_flat = _deg_cnt(edata, attr2, batch_p, z1)
    deg = deg_flat[:NP] + deg_flat[NP:]
    rsq = lax.rsqrt(jnp.maximum(deg, 1e-6)).reshape(NP, 1)
    cnt = cnt_flat[:G] + cnt_flat[G:]

    hs1 = _tc1(x_p, W_enc, b_enc.astype(f32).reshape(1, H), W1, rsq)
    agg1 = _agg(hs1, edata, arep, z2).reshape(2, NP, H)
    hs2 = _tc2(agg1, rsq, b1.astype(f32).reshape(1, H), W2)
    agg2 = _agg(hs2, edata, arep, z2).reshape(2, NP, H)
    out2 = _tc3(agg2, rsq, b2.astype(f32).reshape(1, H))
    sums_p = _pool(out2, batch_p, z2)
    sums = sums_p[:G] + sums_p[G:]
    return sums / jnp.maximum(cnt, 1.0)[:, None]


# speedup vs baseline: 1.0381x; 1.0381x over previous
import functools

import jax
import jax.numpy as jnp
from jax import lax
from jax.experimental import pallas as pl
from jax.experimental.pallas import tpu as pltpu
from jax.experimental.pallas import tpu_sc as plsc

N = 10000
NP = 10240
E = 320000
EP = 322560
H = 128
G = 128
C = 96
EPT = EP // 32
EGROUPS = EPT // C
NGTOT = EP // C
RT = NP // 16
NT = NP // 32
NC = 2
GA = 2 * G
KB = 3
PB = 4

_mesh = plsc.VectorSubcoreMesh(core_axis_name="c", subcore_axis_name="s")


def _wid():
    return lax.axis_index("s") * NC + lax.axis_index("c")


@functools.partial(
    pl.kernel,
    mesh=_mesh,
    out_type=[
        jax.ShapeDtypeStruct((2 * NP,), jnp.float32),
        jax.ShapeDtypeStruct((2 * G,), jnp.float32),
    ],
    scratch_types=[
        pltpu.VMEM((2, 2, C), jnp.int32),
        pltpu.VMEM((2, C), jnp.float32),
        pltpu.VMEM((64,), jnp.int32),
        pltpu.VMEM((64,), jnp.float32),
        pltpu.VMEM_SHARED((NP,), jnp.float32),
        pltpu.VMEM_SHARED((GA,), jnp.float32),
        pltpu.SemaphoreType.DMA((2,)),
        pltpu.SemaphoreType.DMA((2,)),
        pltpu.SemaphoreType.DMA((2,)),
    ],
)
def _deg_cnt(edata_h, attr2_h, batch_h, z1_h, deg_out, cnt_out,
             ebuf, abuf, nidx_v, nval_v, deg_acc, cnt_acc, esem, asem, ssem):
    cid = lax.axis_index("c")
    sid = lax.axis_index("s")
    wid = _wid()
    pltpu.sync_copy(z1_h, deg_acc.at[pl.ds(sid * RT, RT)])

    @pl.when(sid == 0)
    def _():
        pltpu.sync_copy(z1_h.at[pl.ds(0, GA)], cnt_acc)

    plsc.subcore_barrier()

    def dissue(g, b):
        pltpu.async_copy(edata_h.at[wid * EGROUPS + g], ebuf.at[b], esem.at[b])
        pltpu.async_copy(attr2_h.at[wid * EGROUPS + g], abuf.at[b], asem.at[b])

    dissue(0, 0)

    def ebody(g, carry):
        b = lax.rem(g, 2)
        nb = 1 - b

        @pl.when(g + 1 < EGROUPS)
        def _():
            @pl.when(g >= 1)
            def _():
                pltpu.make_async_copy(
                    abuf.at[nb], deg_acc.at[ebuf.at[nb, 1]], ssem.at[nb]
                ).wait()
            dissue(g + 1, nb)

        pltpu.make_async_copy(edata_h.at[wid * EGROUPS + g],
                              ebuf.at[b], esem.at[b]).wait()
        pltpu.make_async_copy(attr2_h.at[wid * EGROUPS + g],
                              abuf.at[b], asem.at[b]).wait()
        pltpu.async_copy(abuf.at[b], deg_acc.at[ebuf.at[b, 1]],
                         ssem.at[b], add=True)
        return carry

    lax.fori_loop(0, EGROUPS, ebody, 0)
    for gl in (EGROUPS - 2, EGROUPS - 1):
        b = gl % 2
        pltpu.make_async_copy(abuf.at[b], deg_acc.at[ebuf.at[b, 1]],
                              ssem.at[b]).wait()

    for k in range(64 // 16):
        nval_v[pl.ds(k * 16, 16)] = jnp.full((16,), 1.0, jnp.float32)
    for g in range(NT // 64):
        nbase = wid * NT + g * 64
        pltpu.sync_copy(batch_h.at[pl.ds(nbase, 64)], nidx_v)
        pltpu.sync_copy(nval_v, cnt_acc.at[nidx_v], add=True)

    plsc.subcore_barrier()
    pltpu.sync_copy(deg_acc.at[pl.ds(sid * RT, RT)],
                    deg_out.at[pl.ds(cid * NP + sid * RT, RT)])

    @pl.when(sid == 0)
    def _():
        pltpu.sync_copy(cnt_acc.at[pl.ds(0, G)], cnt_out.at[pl.ds(cid * G, G)])


@functools.partial(
    pl.kernel,
    mesh=_mesh,
    out_type=jax.ShapeDtypeStruct((2 * NP, H), jnp.float32),
    scratch_types=[
        pltpu.VMEM((PB, 2, C), jnp.int32),
        pltpu.VMEM((PB, C * 16), jnp.float32),
        pltpu.VMEM((KB, C, H), jnp.float32),
        pltpu.VMEM_SHARED((NP, H), jnp.float32),
        pltpu.SemaphoreType.DMA((PB,)),
        pltpu.SemaphoreType.DMA((PB,)),
        pltpu.SemaphoreType.DMA((KB,)),
        pltpu.SemaphoreType.DMA((KB,)),
    ],
)
def _agg(hs_h, edata_h, arep_h, z2_h, out_h,
         ebuf, awbuf, rows_v, acc, esem, wsem, gsem, ssem):
    cid = lax.axis_index("c")
    sid = lax.axis_index("s")
    wid = _wid()
    pltpu.sync_copy(z2_h, acc.at[pl.ds(sid * RT, RT)])
    plsc.subcore_barrier()

    def fetch(g, m):
        pltpu.async_copy(edata_h.at[wid * EGROUPS + g], ebuf.at[m], esem.at[m])
        pltpu.async_copy(arep_h.at[wid * EGROUPS + g], awbuf.at[m], wsem.at[m])

    def wait_fetch_e(g, m):
        pltpu.make_async_copy(edata_h.at[wid * EGROUPS + g], ebuf.at[m],
                              esem.at[m]).wait()

    def gather(m, r):
        pltpu.async_copy(hs_h.at[ebuf.at[m, 0]], rows_v.at[r], gsem.at[r])

    def wait_scatter(m, r):
        pltpu.make_async_copy(rows_v.at[r], acc.at[ebuf.at[m, 1]],
                              ssem.at[r]).wait()

    for g in range(3):
        fetch(g, g)
    for g in range(2):
        wait_fetch_e(g, g)
        gather(g, g)

    def body(g, carry):
        b3 = lax.rem(g, KB)
        b5 = lax.rem(g, PB)
        nb3 = lax.rem(g + 2, KB)
        pb5 = lax.rem(g + 2, PB)
        fb5 = lax.rem(g + 3, PB)

        pltpu.make_async_copy(arep_h.at[wid * EGROUPS + g], awbuf.at[b5],
                              wsem.at[b5]).wait()
        pltpu.make_async_copy(hs_h.at[ebuf.at[b5, 0]], rows_v.at[b3],
                              gsem.at[b3]).wait()

        def _mul(e, mc):
            w16 = awbuf[b5, pl.ds(e * 16, 16)]
            for j in range(H // 16):
                sl = pl.ds(j * 16, 16)
                rows_v[b3, e, sl] = rows_v[b3, e, sl] * w16
            return mc

        lax.fori_loop(0, C, _mul, 0, unroll=16)
        pltpu.async_copy(rows_v.at[b3], acc.at[ebuf.at[b5, 1]],
                         ssem.at[b3], add=True)

        @pl.when(g + 2 < EGROUPS)
        def _():
            @pl.when(g >= 1)
            def _():
                wait_scatter(pb5, nb3)
            wait_fetch_e(g + 2, pb5)
            gather(pb5, nb3)

        @pl.when(g + 3 < EGROUPS)
        def _():
            fetch(g + 3, fb5)

        return carry

    lax.fori_loop(0, EGROUPS, body, 0)
    for gl in (EGROUPS - 3, EGROUPS - 2, EGROUPS - 1):
        wait_scatter(gl % PB, gl % KB)
    plsc.subcore_barrier()
    pltpu.sync_copy(acc.at[pl.ds(sid * RT, RT)],
                    out_h.at[pl.ds(cid * NP + sid * RT, RT)])


@functools.partial(
    pl.kernel,
    mesh=_mesh,
    out_type=jax.ShapeDtypeStruct((2 * G, H), jnp.float32),
    scratch_types=[
        pltpu.VMEM((64,), jnp.int32),
        pltpu.VMEM((64, H), jnp.float32),
        pltpu.VMEM_SHARED((GA, H), jnp.float32),
    ],
)
def _pool(x_h, batch_h, z2_h, out_h, nidx_v, rows_v, acc):
    cid = lax.axis_index("c")
    sid = lax.axis_index("s")
    wid = _wid()

    @pl.when(sid == 0)
    def _():
        pltpu.sync_copy(z2_h.at[pl.ds(0, GA)], acc)

    plsc.subcore_barrier()
    for g in range(NT // 64):
        nbase = wid * NT + g * 64
        pltpu.sync_copy(batch_h.at[pl.ds(nbase, 64)], nidx_v)
        pltpu.sync_copy(x_h.at[pl.ds(nbase, 64)], rows_v)
        pltpu.sync_copy(rows_v, acc.at[nidx_v], add=True)
    plsc.subcore_barrier()

    @pl.when(sid == 0)
    def _():
        pltpu.sync_copy(acc.at[pl.ds(0, G)], out_h.at[pl.ds(cid * G, G)])


_B = 512
_GRID = NP // _B


def _tc1_body(x_ref, we_ref, be_ref, w1_ref, rsq_ref, o_ref):
    h0 = jnp.dot(x_ref[...], we_ref[...],
                 preferred_element_type=jnp.float32) + be_ref[...]
    hw = jnp.dot(h0, w1_ref[...], preferred_element_type=jnp.float32)
    o_ref[...] = hw * rsq_ref[...]


def _tc2_body(p_ref, rsq_ref, b1_ref, w2_ref, o_ref):
    p = p_ref[...]
    s = (p[0] + p[1]) * rsq_ref[...] + b1_ref[...]
    h1 = jnp.maximum(s, 0.0)
    o_ref[...] = jnp.dot(h1, w2_ref[...],
                         preferred_element_type=jnp.float32) * rsq_ref[...]


def _tc3_body(p_ref, rsq_ref, b2_ref, o_ref):
    p = p_ref[...]
    s = (p[0] + p[1]) * rsq_ref[...] + b2_ref[...]
    o = jnp.maximum(s, 0.0)
    row = (pl.program_id(0) * _B
           + lax.broadcasted_iota(jnp.int32, (_B, 1), 0))
    o_ref[...] = jnp.where(row < N, o, 0.0)


def _tc1(x_p, W_enc, b_enc, W1, rsq):
    return pl.pallas_call(
        _tc1_body,
        grid=(_GRID,),
        in_specs=[
            pl.BlockSpec((_B, H), lambda g: (g, 0)),
            pl.BlockSpec((H, H), lambda g: (0, 0)),
            pl.BlockSpec((1, H), lambda g: (0, 0)),
            pl.BlockSpec((H, H), lambda g: (0, 0)),
            pl.BlockSpec((_B, 1), lambda g: (g, 0)),
        ],
        out_specs=pl.BlockSpec((_B, H), lambda g: (g, 0)),
        out_shape=jax.ShapeDtypeStruct((NP, H), jnp.float32),
    )(x_p, W_enc, b_enc, W1, rsq)


def _tc2(p, rsq, b1, W2):
    return pl.pallas_call(
        _tc2_body,
        grid=(_GRID,),
        in_specs=[
            pl.BlockSpec((2, _B, H), lambda g: (0, g, 0)),
            pl.BlockSpec((_B, 1), lambda g: (g, 0)),
            pl.BlockSpec((1, H), lambda g: (0, 0)),
            pl.BlockSpec((H, H), lambda g: (0, 0)),
        ],
        out_specs=pl.BlockSpec((_B, H), lambda g: (g, 0)),
        out_shape=jax.ShapeDtypeStruct((NP, H), jnp.float32),
    )(p, rsq, b1, W2)


def _tc3(p, rsq, b2):
    return pl.pallas_call(
        _tc3_body,
        grid=(_GRID,),
        in_specs=[
            pl.BlockSpec((2, _B, H), lambda g: (0, g, 0)),
            pl.BlockSpec((_B, 1), lambda g: (g, 0)),
            pl.BlockSpec((1, H), lambda g: (0, 0)),
        ],
        out_specs=pl.BlockSpec((_B, H), lambda g: (g, 0)),
        out_shape=jax.ShapeDtypeStruct((NP, H), jnp.float32),
    )(p, rsq, b2)


def kernel(x, edge_index, edge_attr, batch, W_enc, b_enc, W1, b1, W2, b2):
    f32 = jnp.float32
    i32 = jnp.int32
    src_p = jnp.concatenate([edge_index[0].astype(i32),
                             jnp.zeros((EP - E,), i32)])
    dst_p = jnp.concatenate([edge_index[1].astype(i32),
                             jnp.zeros((EP - E,), i32)])
    attr_p = jnp.concatenate([edge_attr.astype(f32), jnp.zeros((EP - E,), f32)])
    edata = jnp.stack(
        [src_p.reshape(NGTOT, C), dst_p.reshape(NGTOT, C)], axis=1)
    attr2 = attr_p.reshape(NGTOT, C)
    arep = jnp.broadcast_to(attr_p[:, None], (EP, 16)).reshape(NGTOT, C * 16)
    batch_p = jnp.concatenate([batch.astype(i32),
                               jnp.full((NP - N,), G, i32)])
    x_p = jnp.pad(x.astype(f32), ((0, NP - N), (0, 0)))
    z1 = jnp.zeros((RT,), f32)
    z2 = jnp.zeros((RT, H), f32)

    deg_flat, cnt_flat = _deg_cnt(edata, attr2, batch_p, z1)
    deg = deg_flat[:NP] + deg_flat[NP:]
    rsq = lax.rsqrt(jnp.maximum(deg, 1e-6)).reshape(NP, 1)
    cnt = cnt_flat[:G] + cnt_flat[G:]

    hs1 = _tc1(x_p, W_enc, b_enc.astype(f32).reshape(1, H), W1, rsq)
    agg1 = _agg(hs1, edata, arep, z2).reshape(2, NP, H)
    hs2 = _tc2(agg1, rsq, b1.astype(f32).reshape(1, H), W2)
    agg2 = _agg(hs2, edata, arep, z2).reshape(2, NP, H)
    out2 = _tc3(agg2, rsq, b2.astype(f32).reshape(1, H))
    sums_p = _pool(out2, batch_p, z2)
    sums = sums_p[:G] + sums_p[G:]
    return sums / jnp.maximum(cnt, 1.0)[:, None]
